# baseline (speedup 1.0000x reference)
import functools

import jax
import jax.numpy as jnp
from jax import lax
from jax.experimental import pallas as pl
from jax.experimental.pallas import tpu as pltpu

_TEMP = 50.0
_MAX_EFF_TEMP = 5000.0


def _fcm_body(x_ref, yq_ref, yqh_ref, b_ref, choice_ref, v_ref, s_ref, *, nt):
    bs, d = x_ref.shape
    k = yq_ref.shape[0]
    tk = k // nt
    x = x_ref[...]

    m = None
    mn = None
    for t in range(nt):
        yq_t = yq_ref[pl.ds(t * tk, tk), :]
        s_t = lax.dot_general(
            x, yq_t, (((1,), (1,)), ((), ())),
            preferred_element_type=jnp.float32,
        ) + b_ref[:, pl.ds(t * tk, tk)]
        s_ref[:, pl.ds(t * tk, tk)] = s_t
        m_t = jnp.max(s_t, axis=1, keepdims=True)
        mn_t = jnp.min(s_t, axis=1, keepdims=True)
        m = m_t if m is None else jnp.maximum(m, m_t)
        mn = mn_t if mn is None else jnp.minimum(mn, mn_t)

    span = jnp.maximum(m - mn, 1e-3)
    eff = jnp.clip(_TEMP / span, _TEMP, _MAX_EFF_TEMP)
    c = m * eff

    denom = jnp.zeros((bs, 1), jnp.float32)
    ve = jnp.zeros((bs, 1), jnp.float32)
    acc = jnp.zeros((bs, d), jnp.float32)
    for t in range(nt):
        s_t = s_ref[:, pl.ds(t * tk, tk)]
        e_t = jnp.exp(s_t * eff - c).astype(jnp.bfloat16)
        ef_t = e_t.astype(jnp.float32)
        denom = denom + jnp.sum(ef_t, axis=1, keepdims=True)
        ve = ve + jnp.sum(ef_t * s_t, axis=1, keepdims=True)
        acc = acc + jnp.dot(
            e_t, yqh_ref[pl.ds(t * tk, tk), :],
            preferred_element_type=jnp.float32,
        )

    inv = 1.0 / denom
    v_ref[...] = ve * inv
    choice_ref[...] = acc * inv


@functools.partial(jax.jit, static_argnames=("block_s", "nt"))
def _fcm(X, Y, intercept, block_s=256, nt=8):
    S, d = X.shape
    K = Y.shape[1]
    yq = Y[0]
    yqh = yq.astype(jnp.bfloat16)
    grid = (S // block_s,)
    choice, v = pl.pallas_call(
        functools.partial(_fcm_body, nt=nt),
        grid=grid,
        in_specs=[
            pl.BlockSpec((block_s, d), lambda i: (i, 0)),
            pl.BlockSpec((K, d), lambda i: (0, 0)),
            pl.BlockSpec((K, d), lambda i: (0, 0)),
            pl.BlockSpec((1, K), lambda i: (0, 0)),
        ],
        out_specs=[
            pl.BlockSpec((block_s, d), lambda i: (i, 0)),
            pl.BlockSpec((block_s, 1), lambda i: (i, 0)),
        ],
        out_shape=[
            jax.ShapeDtypeStruct((S, d), jnp.float32),
            jax.ShapeDtypeStruct((S, 1), jnp.float32),
        ],
        scratch_shapes=[pltpu.VMEM((block_s, K), jnp.float32)],
    )(X, yq, yqh, intercept)
    return choice, v[:, 0]


def kernel(X, Y, intercept):
    return _fcm(X, Y, intercept)


# speedup vs baseline: 1.0327x; 1.0327x over previous
import functools

import jax
import jax.numpy as jnp
from jax import lax
from jax.experimental import pallas as pl
from jax.experimental.pallas import tpu as pltpu

_TEMP = 50.0
_MAX_EFF_TEMP = 5000.0


def _fcm_body(x_ref, yq_ref, yqh_ref, b_ref, choice_ref, v_ref, s_ref, *, nt):
    bs, d = x_ref.shape
    k = yq_ref.shape[0]
    tk = k // nt
    x = x_ref[...]

    m = None
    mn = None
    for t in range(nt):
        yq_t = yq_ref[pl.ds(t * tk, tk), :]
        s_t = lax.dot_general(
            x, yq_t, (((1,), (1,)), ((), ())),
            preferred_element_type=jnp.float32,
        ) + b_ref[:, pl.ds(t * tk, tk)]
        s_ref[:, pl.ds(t * tk, tk)] = s_t
        m_t = jnp.max(s_t, axis=1, keepdims=True)
        mn_t = jnp.min(s_t, axis=1, keepdims=True)
        m = m_t if m is None else jnp.maximum(m, m_t)
        mn = mn_t if mn is None else jnp.minimum(mn, mn_t)

    span = jnp.maximum(m - mn, 1e-3)
    eff = jnp.clip(_TEMP / span, _TEMP, _MAX_EFF_TEMP)
    c = m * eff

    denom = jnp.zeros((bs, 1), jnp.float32)
    ve = jnp.zeros((bs, 1), jnp.float32)
    acc = jnp.zeros((bs, d), jnp.float32)
    for t in range(nt):
        s_t = s_ref[:, pl.ds(t * tk, tk)]
        e_t = jnp.exp(s_t * eff - c)
        denom = denom + jnp.sum(e_t, axis=1, keepdims=True)
        ve = ve + jnp.sum(e_t * s_t, axis=1, keepdims=True)
        acc = acc + jnp.dot(
            e_t.astype(jnp.bfloat16), yqh_ref[pl.ds(t * tk, tk), :],
            preferred_element_type=jnp.float32,
        )

    inv = 1.0 / denom
    v_ref[...] = ve * inv
    choice_ref[...] = acc * inv


@functools.partial(jax.jit, static_argnames=("block_s", "nt"))
def _fcm(X, Y, intercept, block_s=256, nt=8):
    S, d = X.shape
    K = Y.shape[1]
    yq = Y[0]
    yqh = yq.astype(jnp.bfloat16)
    grid = (S // block_s,)
    choice, v = pl.pallas_call(
        functools.partial(_fcm_body, nt=nt),
        grid=grid,
        in_specs=[
            pl.BlockSpec((block_s, d), lambda i: (i, 0)),
            pl.BlockSpec((K, d), lambda i: (0, 0)),
            pl.BlockSpec((K, d), lambda i: (0, 0)),
            pl.BlockSpec((1, K), lambda i: (0, 0)),
        ],
        out_specs=[
            pl.BlockSpec((block_s, d), lambda i: (i, 0)),
            pl.BlockSpec((block_s, 1), lambda i: (i, 0)),
        ],
        out_shape=[
            jax.ShapeDtypeStruct((S, d), jnp.float32),
            jax.ShapeDtypeStruct((S, 1), jnp.float32),
        ],
        scratch_shapes=[pltpu.VMEM((block_s, K), jnp.float32)],
    )(X, yq, yqh, intercept)
    return choice, v[:, 0]


def kernel(X, Y, intercept):
    return _fcm(X, Y, intercept)


# speedup vs baseline: 1.2231x; 1.1843x over previous
import functools

import jax
import jax.numpy as jnp
from jax import lax
from jax.experimental import pallas as pl
from jax.experimental.pallas import tpu as pltpu

_TEMP = 50.0
_MAX_EFF_TEMP = 5000.0


def _fcm_body(x_ref, yq_ref, aug_ref, b_ref, choice_ref, v_ref, s_ref, *, nt):
    bs, d = x_ref.shape
    k = yq_ref.shape[0]
    tk = k // nt
    x = x_ref[...]

    m = None
    mn = None
    for t in range(nt):
        yq_t = yq_ref[pl.ds(t * tk, tk), :]
        s_t = lax.dot_general(
            x, yq_t, (((1,), (1,)), ((), ())),
            preferred_element_type=jnp.float32,
        ) + b_ref[:, pl.ds(t * tk, tk)]
        s_ref[:, pl.ds(t * tk, tk)] = s_t
        m_t = jnp.max(s_t, axis=1, keepdims=True)
        mn_t = jnp.min(s_t, axis=1, keepdims=True)
        m = m_t if m is None else jnp.maximum(m, m_t)
        mn = mn_t if mn is None else jnp.minimum(mn, mn_t)

    span = jnp.maximum(m - mn, 1e-3)
    eff = jnp.clip(_TEMP / span, _TEMP, _MAX_EFF_TEMP)
    c = m * eff

    acc = jnp.zeros((bs, aug_ref.shape[1]), jnp.float32)
    for t in range(nt):
        s_t = s_ref[:, pl.ds(t * tk, tk)]
        e_t = jnp.exp(s_t * eff - c)
        acc = acc + jnp.dot(
            e_t, aug_ref[pl.ds(t * tk, tk), :],
            preferred_element_type=jnp.float32,
        )

    cacc = acc[:, :d]
    wb = acc[:, d:d + 1]
    denom = acc[:, d + 1:d + 2]
    inv = 1.0 / denom
    choice_ref[...] = cacc * inv
    v_ref[...] = (jnp.sum(x * cacc, axis=1, keepdims=True) + wb) * inv


@functools.partial(jax.jit, static_argnames=("block_s", "nt"))
def _fcm(X, Y, intercept, block_s=256, nt=8):
    S, d = X.shape
    K = Y.shape[1]
    yq = Y[0]
    aug = jnp.concatenate(
        [yq, intercept.T, jnp.ones((K, 1), jnp.float32)], axis=1)
    grid = (S // block_s,)
    choice, v = pl.pallas_call(
        functools.partial(_fcm_body, nt=nt),
        grid=grid,
        in_specs=[
            pl.BlockSpec((block_s, d), lambda i: (i, 0)),
            pl.BlockSpec((K, d), lambda i: (0, 0)),
            pl.BlockSpec((K, d + 2), lambda i: (0, 0)),
            pl.BlockSpec((1, K), lambda i: (0, 0)),
        ],
        out_specs=[
            pl.BlockSpec((block_s, d), lambda i: (i, 0)),
            pl.BlockSpec((block_s, 1), lambda i: (i, 0)),
        ],
        out_shape=[
            jax.ShapeDtypeStruct((S, d), jnp.float32),
            jax.ShapeDtypeStruct((S, 1), jnp.float32),
        ],
        scratch_shapes=[pltpu.VMEM((block_s, K), jnp.float32)],
    )(X, yq, aug, intercept)
    return choice, v[:, 0]


def kernel(X, Y, intercept):
    return _fcm(X, Y, intercept)


# speedup vs baseline: 1.2287x; 1.0046x over previous
import functools

import jax
import jax.numpy as jnp
from jax import lax
from jax.experimental import pallas as pl
from jax.experimental.pallas import tpu as pltpu

_TEMP = 50.0
_MAX_EFF_TEMP = 5000.0


def _fcm_body(x_ref, yq_ref, aug_ref, b_ref, choice_ref, v_ref, s_ref, *, nt):
    bs, d = x_ref.shape
    k = yq_ref.shape[0]
    tk = k // nt
    x = x_ref[...]

    m = None
    mn = None
    for t in range(nt):
        yq_t = yq_ref[pl.ds(t * tk, tk), :]
        s_t = lax.dot_general(
            x, yq_t, (((1,), (1,)), ((), ())),
            preferred_element_type=jnp.float32,
        ) + b_ref[:, pl.ds(t * tk, tk)]
        s_ref[:, pl.ds(t * tk, tk)] = s_t
        m_t = jnp.max(s_t, axis=1, keepdims=True)
        mn_t = jnp.min(s_t, axis=1, keepdims=True)
        m = m_t if m is None else jnp.maximum(m, m_t)
        mn = mn_t if mn is None else jnp.minimum(mn, mn_t)

    span = jnp.maximum(m - mn, 1e-3)
    eff = jnp.clip(_TEMP / span, _TEMP, _MAX_EFF_TEMP)
    eff2 = eff * jnp.float32(1.4426950408889634)
    c2 = m * eff2

    acc = jnp.zeros((bs, aug_ref.shape[1]), jnp.float32)
    for t in range(nt):
        s_t = s_ref[:, pl.ds(t * tk, tk)]
        e_t = jnp.exp2(s_t * eff2 - c2)
        acc = acc + jnp.dot(
            e_t, aug_ref[pl.ds(t * tk, tk), :],
            preferred_element_type=jnp.float32,
        )

    cacc = acc[:, :d]
    wb = acc[:, d:d + 1]
    denom = acc[:, d + 1:d + 2]
    inv = 1.0 / denom
    choice_ref[...] = cacc * inv
    v_ref[...] = (jnp.sum(x * cacc, axis=1, keepdims=True) + wb) * inv


@functools.partial(jax.jit, static_argnames=("block_s", "nt"))
def _fcm(X, Y, intercept, block_s=256, nt=8):
    S, d = X.shape
    K = Y.shape[1]
    yq = Y[0]
    aug = jnp.concatenate(
        [yq, intercept.T, jnp.ones((K, 1), jnp.float32)], axis=1)
    grid = (S // block_s,)
    choice, v = pl.pallas_call(
        functools.partial(_fcm_body, nt=nt),
        grid=grid,
        in_specs=[
            pl.BlockSpec((block_s, d), lambda i: (i, 0)),
            pl.BlockSpec((K, d), lambda i: (0, 0)),
            pl.BlockSpec((K, d + 2), lambda i: (0, 0)),
            pl.BlockSpec((1, K), lambda i: (0, 0)),
        ],
        out_specs=[
            pl.BlockSpec((block_s, d), lambda i: (i, 0)),
            pl.BlockSpec((block_s, 1), lambda i: (i, 0)),
        ],
        out_shape=[
            jax.ShapeDtypeStruct((S, d), jnp.float32),
            jax.ShapeDtypeStruct((S, 1), jnp.float32),
        ],
        scratch_shapes=[pltpu.VMEM((block_s, K), jnp.float32)],
    )(X, yq, aug, intercept)
    return choice, v[:, 0]


def kernel(X, Y, intercept):
    return _fcm(X, Y, intercept)


# speedup vs baseline: 1.2847x; 1.0456x over previous
import functools

import jax
import jax.numpy as jnp
from jax import lax
from jax.experimental import pallas as pl
from jax.experimental.pallas import tpu as pltpu

_TEMP = 50.0
_MAX_EFF_TEMP = 5000.0


def _fcm_body(x_ref, yq_ref, aug_ref, b_ref, choice_ref, v_ref, s_ref, *, nt):
    bs, d = x_ref.shape
    k = yq_ref.shape[0]
    tk = k // nt
    x = x_ref[...]

    m = None
    mn = None
    for t in range(nt):
        yq_t = yq_ref[pl.ds(t * tk, tk), :]
        s_t = lax.dot_general(
            x, yq_t, (((1,), (1,)), ((), ())),
            preferred_element_type=jnp.float32,
        ) + b_ref[:, pl.ds(t * tk, tk)]
        s_ref[:, pl.ds(t * tk, tk)] = s_t
        m_t = jnp.max(s_t, axis=1, keepdims=True)
        mn_t = jnp.min(s_t, axis=1, keepdims=True)
        m = m_t if m is None else jnp.maximum(m, m_t)
        mn = mn_t if mn is None else jnp.minimum(mn, mn_t)

    span = jnp.maximum(m - mn, 1e-3)
    eff = jnp.clip(_TEMP / span, _TEMP, _MAX_EFF_TEMP)
    eff2 = eff * jnp.float32(1.4426950408889634)
    c2 = m * eff2

    acc = jnp.zeros((bs, aug_ref.shape[1]), jnp.float32)
    for t in range(nt):
        s_t = s_ref[:, pl.ds(t * tk, tk)]
        e_t = jnp.exp2(s_t * eff2 - c2)
        acc = acc + jnp.dot(
            e_t, aug_ref[pl.ds(t * tk, tk), :],
            preferred_element_type=jnp.float32,
        )

    cacc = acc[:, :d]
    wb = acc[:, d:d + 1]
    denom = acc[:, d + 1:d + 2]
    inv = 1.0 / denom
    choice_ref[...] = cacc * inv
    v_ref[...] = (jnp.sum(x * cacc, axis=1, keepdims=True) + wb) * inv


@functools.partial(jax.jit, static_argnames=("block_s", "nt"))
def _fcm(X, Y, intercept, block_s=512, nt=8):
    S, d = X.shape
    K = Y.shape[1]
    yq = Y[0]
    aug = jnp.concatenate(
        [yq, intercept.T, jnp.ones((K, 1), jnp.float32)], axis=1)
    grid = (S // block_s,)
    choice, v = pl.pallas_call(
        functools.partial(_fcm_body, nt=nt),
        grid=grid,
        in_specs=[
            pl.BlockSpec((block_s, d), lambda i: (i, 0)),
            pl.BlockSpec((K, d), lambda i: (0, 0)),
            pl.BlockSpec((K, d + 2), lambda i: (0, 0)),
            pl.BlockSpec((1, K), lambda i: (0, 0)),
        ],
        out_specs=[
            pl.BlockSpec((block_s, d), lambda i: (i, 0)),
            pl.BlockSpec((block_s, 1), lambda i: (i, 0)),
        ],
        out_shape=[
            jax.ShapeDtypeStruct((S, d), jnp.float32),
            jax.ShapeDtypeStruct((S, 1), jnp.float32),
        ],
        scratch_shapes=[pltpu.VMEM((block_s, K), jnp.float32)],
    )(X, yq, aug, intercept)
    return choice, v[:, 0]


def kernel(X, Y, intercept):
    return _fcm(X, Y, intercept)


# speedup vs baseline: 1.3077x; 1.0179x over previous
import functools

import jax
import jax.numpy as jnp
from jax import lax
from jax.experimental import pallas as pl
from jax.experimental.pallas import tpu as pltpu

_TEMP = 50.0
_MAX_EFF_TEMP = 5000.0
_LOG2E = 1.4426950408889634


def _fcm_body(x_ref, yq_ref, aug_ref, b_ref, choice_ref, v_ref,
              s_ref, e2_ref, c2_ref, xp_ref, *, nt):
    bs, d = x_ref.shape
    k = yq_ref.shape[0]
    tk = k // nt
    i = pl.program_id(0)
    p = lax.rem(i, 2)
    q = 1 - p

    x = x_ref[...]
    xp_ref[p] = x
    m = None
    mn = None
    for t in range(nt):
        yq_t = yq_ref[pl.ds(t * tk, tk), :]
        s_t = lax.dot_general(
            x, yq_t, (((1,), (1,)), ((), ())),
            preferred_element_type=jnp.float32,
        ) + b_ref[:, pl.ds(t * tk, tk)]
        s_ref[p, :, pl.ds(t * tk, tk)] = s_t
        m_t = jnp.max(s_t, axis=1, keepdims=True)
        mn_t = jnp.min(s_t, axis=1, keepdims=True)
        m = m_t if m is None else jnp.maximum(m, m_t)
        mn = mn_t if mn is None else jnp.minimum(mn, mn_t)
    span = jnp.maximum(m - mn, 1e-3)
    eff = jnp.clip(_TEMP / span, _TEMP, _MAX_EFF_TEMP)
    eff2 = eff * jnp.float32(_LOG2E)
    e2_ref[p] = eff2
    c2_ref[p] = m * eff2

    eff2q = e2_ref[q]
    c2q = c2_ref[q]
    acc = jnp.zeros((bs, aug_ref.shape[1]), jnp.float32)
    for t in range(nt):
        s_t = s_ref[q, :, pl.ds(t * tk, tk)]
        e_t = jnp.exp2(s_t * eff2q - c2q)
        acc = acc + jnp.dot(
            e_t, aug_ref[pl.ds(t * tk, tk), :],
            preferred_element_type=jnp.float32,
        )
    cacc = acc[:, :d]
    wb = acc[:, d:d + 1]
    denom = acc[:, d + 1:d + 2]
    inv = 1.0 / denom
    choice_ref[...] = cacc * inv
    xq = xp_ref[q]
    v_ref[...] = (jnp.sum(xq * cacc, axis=1, keepdims=True) + wb) * inv


@functools.partial(jax.jit, static_argnames=("block_s", "nt"))
def _fcm(X, Y, intercept, block_s=256, nt=8):
    S, d = X.shape
    K = Y.shape[1]
    nb = S // block_s
    yq = Y[0]
    aug = jnp.concatenate(
        [yq, intercept.T, jnp.ones((K, 1), jnp.float32)], axis=1)
    grid = (nb + 1,)
    choice, v = pl.pallas_call(
        functools.partial(_fcm_body, nt=nt),
        grid=grid,
        in_specs=[
            pl.BlockSpec((block_s, d), lambda i: (jnp.minimum(i, nb - 1), 0)),
            pl.BlockSpec((K, d), lambda i: (0, 0)),
            pl.BlockSpec((K, d + 2), lambda i: (0, 0)),
            pl.BlockSpec((1, K), lambda i: (0, 0)),
        ],
        out_specs=[
            pl.BlockSpec((block_s, d), lambda i: (jnp.maximum(i - 1, 0), 0)),
            pl.BlockSpec((block_s, 1), lambda i: (jnp.maximum(i - 1, 0), 0)),
        ],
        out_shape=[
            jax.ShapeDtypeStruct((S, d), jnp.float32),
            jax.ShapeDtypeStruct((S, 1), jnp.float32),
        ],
        scratch_shapes=[
            pltpu.VMEM((2, block_s, K), jnp.float32),
            pltpu.VMEM((2, block_s, 1), jnp.float32),
            pltpu.VMEM((2, block_s, 1), jnp.float32),
            pltpu.VMEM((2, block_s, d), jnp.float32),
        ],
    )(X, yq, aug, intercept)
    return choice, v[:, 0]


def kernel(X, Y, intercept):
    return _fcm(X, Y, intercept)


# speedup vs baseline: 1.3492x; 1.0317x over previous
import functools

import jax
import jax.numpy as jnp
from jax import lax
from jax.experimental import pallas as pl
from jax.experimental.pallas import tpu as pltpu

_TEMP = 50.0
_MAX_EFF_TEMP = 5000.0
_LOG2E = 1.4426950408889634


def _fcm_body(x_ref, aug_ref, b_ref, choice_ref, v_ref,
              s_ref, e2_ref, c2_ref, xp_ref, *, nt):
    bs, d = x_ref.shape
    k = aug_ref.shape[0]
    tk = k // nt
    i = pl.program_id(0)
    p = lax.rem(i, 2)
    q = 1 - p

    x = x_ref[...]
    xp_ref[p] = x
    m = None
    mn = None
    for t in range(nt):
        yq_t = aug_ref[pl.ds(t * tk, tk), :d]
        s_t = lax.dot_general(
            x, yq_t, (((1,), (1,)), ((), ())),
            preferred_element_type=jnp.float32,
        ) + b_ref[:, pl.ds(t * tk, tk)]
        s_ref[p, :, pl.ds(t * tk, tk)] = s_t
        m_t = jnp.max(s_t, axis=1, keepdims=True)
        mn_t = jnp.min(s_t, axis=1, keepdims=True)
        m = m_t if m is None else jnp.maximum(m, m_t)
        mn = mn_t if mn is None else jnp.minimum(mn, mn_t)
    span = jnp.maximum(m - mn, 1e-3)
    eff = jnp.clip(_TEMP / span, _TEMP, _MAX_EFF_TEMP)
    eff2 = eff * jnp.float32(_LOG2E)
    e2_ref[p] = eff2
    c2_ref[p] = m * eff2

    eff2q = e2_ref[q]
    c2q = c2_ref[q]
    acc = jnp.zeros((bs, aug_ref.shape[1]), jnp.float32)
    for t in range(nt):
        s_t = s_ref[q, :, pl.ds(t * tk, tk)]
        e_t = jnp.exp2(s_t * eff2q - c2q)
        acc = acc + jnp.dot(
            e_t, aug_ref[pl.ds(t * tk, tk), :],
            preferred_element_type=jnp.float32,
        )
    cacc = acc[:, :d]
    wb = acc[:, d:d + 1]
    denom = acc[:, d + 1:d + 2]
    inv = 1.0 / denom
    choice_ref[...] = cacc * inv
    xq = xp_ref[q]
    v_ref[...] = (jnp.sum(xq * cacc, axis=1, keepdims=True) + wb) * inv


@functools.partial(jax.jit, static_argnames=("block_s", "nt"))
def _fcm(X, Y, intercept, block_s=512, nt=16):
    S, d = X.shape
    K = Y.shape[1]
    nb = S // block_s
    yq = Y[0]
    aug = jnp.concatenate(
        [yq, intercept.T, jnp.ones((K, 1), jnp.float32)], axis=1)
    grid = (nb + 1,)
    choice, v = pl.pallas_call(
        functools.partial(_fcm_body, nt=nt),
        grid=grid,
        in_specs=[
            pl.BlockSpec((block_s, d), lambda i: (jnp.minimum(i, nb - 1), 0)),
            pl.BlockSpec((K, d + 2), lambda i: (0, 0)),
            pl.BlockSpec((1, K), lambda i: (0, 0)),
        ],
        out_specs=[
            pl.BlockSpec((block_s, d), lambda i: (jnp.maximum(i - 1, 0), 0)),
            pl.BlockSpec((block_s, 1), lambda i: (jnp.maximum(i - 1, 0), 0)),
        ],
        out_shape=[
            jax.ShapeDtypeStruct((S, d), jnp.float32),
            jax.ShapeDtypeStruct((S, 1), jnp.float32),
        ],
        scratch_shapes=[
            pltpu.VMEM((2, block_s, K), jnp.float32),
            pltpu.VMEM((2, block_s, 1), jnp.float32),
            pltpu.VMEM((2, block_s, 1), jnp.float32),
            pltpu.VMEM((2, block_s, d), jnp.float32),
        ],
    )(X, aug, intercept)
    return choice, v[:, 0]


def kernel(X, Y, intercept):
    return _fcm(X, Y, intercept)
